# baseline (speedup 1.0000x reference)
import jax
import jax.numpy as jnp
from jax.experimental import pallas as pl

K = 8192
W = 4096
D = 32
R = 256


def _vq_block(x_ref, cb_ref, x2_ref, e2_ref, out_ref):
    x = x_ref[...]
    cb = cb_ref[...]
    x2 = x2_ref[...]
    e2 = e2_ref[...]
    sim = jax.lax.dot_general(
        x, cb, (((1,), (1,)), ((), ())),
        preferred_element_type=jnp.float32,
    )
    d = (x2 + e2) - 2.0 * sim
    lane = jax.lax.broadcasted_iota(jnp.int32, (R, K), 1)
    d1 = d[:, :W]
    d2 = d[:, W:]
    m1 = jnp.min(d1, axis=1, keepdims=True)
    i1 = jnp.min(jnp.where(d1 == m1, lane[:, :W], K), axis=1, keepdims=True)
    m2 = jnp.min(d2, axis=1, keepdims=True)
    i2 = jnp.min(jnp.where(d2 == m2, lane[:, W:], K), axis=1, keepdims=True)
    v1b = (-m1).astype(jnp.bfloat16).astype(jnp.float32)
    idx = jnp.where((-m2) > v1b, i2, i1)
    onehot = (lane == idx).astype(jnp.float32)
    q = jax.lax.dot_general(
        onehot, cb, (((1,), (0,)), ((), ())),
        preferred_element_type=jnp.float32,
    )
    out_ref[...] = x + (q - x)


@jax.jit
def kernel(x, codebook):
    flat = jnp.reshape(x, (-1, D))
    n = flat.shape[0]
    x2 = jnp.sum(jnp.square(flat), axis=1, keepdims=True)
    e2 = jnp.sum(jnp.square(codebook), axis=1)[None, :]
    y = pl.pallas_call(
        _vq_block,
        grid=(n // R,),
        in_specs=[
            pl.BlockSpec((R, D), lambda i: (i, 0)),
            pl.BlockSpec((K, D), lambda i: (0, 0)),
            pl.BlockSpec((R, 1), lambda i: (i, 0)),
            pl.BlockSpec((1, K), lambda i: (0, 0)),
        ],
        out_specs=pl.BlockSpec((R, D), lambda i: (i, 0)),
        out_shape=jax.ShapeDtypeStruct((n, D), jnp.float32),
    )(flat, codebook, x2, e2)
    return jnp.reshape(y, x.shape)


# speedup vs baseline: 1.1485x; 1.1485x over previous
import jax
import jax.numpy as jnp
from jax.experimental import pallas as pl

K = 8192
W = 4096
D = 32
R = 256
T = 512
NT = K // T


def _vq_block(x_ref, cb_ref, x2_ref, e2_ref, out_ref):
    x = x_ref[...]
    x2 = x2_ref[...]
    lane = jax.lax.broadcasted_iota(jnp.int32, (R, T), 1)
    win = []
    for w in range(2):
        md = jnp.full((R, T), jnp.inf, jnp.float32)
        ti = jnp.zeros((R, T), jnp.int32)
        for t in range(w * (W // T), (w + 1) * (W // T)):
            cb_t = cb_ref[t * T:(t + 1) * T, :]
            e2_t = e2_ref[:, t * T:(t + 1) * T]
            sim = jax.lax.dot_general(
                x, cb_t, (((1,), (1,)), ((), ())),
                preferred_element_type=jnp.float32,
            )
            d = (x2 + e2_t) - 2.0 * sim
            ti = jnp.where(d < md, t, ti)
            md = jnp.minimum(md, d)
        ci = ti * T + lane
        m = jnp.min(md, axis=1, keepdims=True)
        i = jnp.min(jnp.where(md == m, ci, K), axis=1, keepdims=True)
        win.append((m, i))
    (m1, i1), (m2, i2) = win
    v1b = (-m1).astype(jnp.bfloat16).astype(jnp.float32)
    idx = jnp.where((-m2) > v1b, i2, i1)
    cb = cb_ref[...]
    cb_hi = cb.astype(jnp.bfloat16)
    cb_lo = (cb - cb_hi.astype(jnp.float32)).astype(jnp.bfloat16)
    klane = jax.lax.broadcasted_iota(jnp.int32, (R, K), 1)
    onehot = (klane == idx).astype(jnp.bfloat16)
    dn = (((1,), (0,)), ((), ()))
    q = (jax.lax.dot_general(onehot, cb_hi, dn, preferred_element_type=jnp.float32)
         + jax.lax.dot_general(onehot, cb_lo, dn, preferred_element_type=jnp.float32))
    out_ref[...] = x + (q - x)


@jax.jit
def kernel(x, codebook):
    flat = jnp.reshape(x, (-1, D))
    n = flat.shape[0]
    x2 = jnp.sum(jnp.square(flat), axis=1, keepdims=True)
    e2 = jnp.sum(jnp.square(codebook), axis=1)[None, :]
    y = pl.pallas_call(
        _vq_block,
        grid=(n // R,),
        in_specs=[
            pl.BlockSpec((R, D), lambda i: (i, 0)),
            pl.BlockSpec((K, D), lambda i: (0, 0)),
            pl.BlockSpec((R, 1), lambda i: (i, 0)),
            pl.BlockSpec((1, K), lambda i: (0, 0)),
        ],
        out_specs=pl.BlockSpec((R, D), lambda i: (i, 0)),
        out_shape=jax.ShapeDtypeStruct((n, D), jnp.float32),
    )(flat, codebook, x2, e2)
    return jnp.reshape(y, x.shape)


# speedup vs baseline: 1.1809x; 1.0283x over previous
import jax
import jax.numpy as jnp
from jax.experimental import pallas as pl

K = 8192
W = 4096
D = 32
R = 256
T = 256
NT = K // T
TSHIFT = 8


def _vq_block(x_ref, cb_ref, x2_ref, e2_ref, cbh_ref, cbl_ref, out_ref):
    x = x_ref[...]
    x2 = x2_ref[...]
    lane = jax.lax.broadcasted_iota(jnp.int32, (R, T), 1)
    win = []
    for w in range(2):
        md = jnp.full((R, T), jnp.inf, jnp.float32)
        ti = jnp.zeros((R, T), jnp.int32)
        for t in range(w * (W // T), (w + 1) * (W // T)):
            cb_t = cb_ref[t * T:(t + 1) * T, :]
            e2_t = e2_ref[:, t * T:(t + 1) * T]
            sim = jax.lax.dot_general(
                x, cb_t, (((1,), (1,)), ((), ())),
                preferred_element_type=jnp.float32,
            )
            d = (x2 + e2_t) - 2.0 * sim
            ti = jnp.where(d < md, t, ti)
            md = jnp.minimum(md, d)
        ci = ti * T + lane
        m = jnp.min(md, axis=1, keepdims=True)
        i = jnp.min(jnp.where(md == m, ci, K), axis=1, keepdims=True)
        win.append((m, i))
    (m1, i1), (m2, i2) = win
    v1b = (-m1).astype(jnp.bfloat16).astype(jnp.float32)
    idx = jnp.where((-m2) > v1b, i2, i1)
    ohl = (lane == (idx & (T - 1))).astype(jnp.bfloat16)
    tid = jax.lax.shift_right_logical(idx, TSHIFT)
    w2 = jax.lax.dot_general(
        ohl, cbh_ref[...], (((1,), (0,)), ((), ())),
        preferred_element_type=jnp.float32,
    ) + jax.lax.dot_general(
        ohl, cbl_ref[...], (((1,), (0,)), ((), ())),
        preferred_element_type=jnp.float32,
    )
    q = jnp.zeros((R, D), jnp.float32)
    for t in range(NT):
        q = jnp.where(tid == t, w2[:, t * D:(t + 1) * D], q)
    out_ref[...] = x + (q - x)


@jax.jit
def kernel(x, codebook):
    flat = jnp.reshape(x, (-1, D))
    n = flat.shape[0]
    x2 = jnp.sum(jnp.square(flat), axis=1, keepdims=True)
    e2 = jnp.sum(jnp.square(codebook), axis=1)[None, :]
    cbr = jnp.reshape(
        jnp.transpose(jnp.reshape(codebook, (NT, T, D)), (1, 0, 2)),
        (T, NT * D))
    cbh = cbr.astype(jnp.bfloat16)
    cbl = (cbr - cbh.astype(jnp.float32)).astype(jnp.bfloat16)
    y = pl.pallas_call(
        _vq_block,
        grid=(n // R,),
        in_specs=[
            pl.BlockSpec((R, D), lambda i: (i, 0)),
            pl.BlockSpec((K, D), lambda i: (0, 0)),
            pl.BlockSpec((R, 1), lambda i: (i, 0)),
            pl.BlockSpec((1, K), lambda i: (0, 0)),
            pl.BlockSpec((T, NT * D), lambda i: (0, 0)),
            pl.BlockSpec((T, NT * D), lambda i: (0, 0)),
        ],
        out_specs=pl.BlockSpec((R, D), lambda i: (i, 0)),
        out_shape=jax.ShapeDtypeStruct((n, D), jnp.float32),
    )(flat, codebook, x2, e2, cbh, cbl)
    return jnp.reshape(y, x.shape)


# speedup vs baseline: 1.4994x; 1.2697x over previous
import jax
import jax.numpy as jnp
from jax.experimental import pallas as pl

K = 8192
W = 4096
D = 32
R = 256
T = 1024
NT = K // T
TSHIFT = 10


def _vq_block(x_ref, cb_ref, x2_ref, e2_ref, cbh_ref, cbl_ref, out_ref):
    x = x_ref[...]
    x2 = x2_ref[...]
    xm2 = x * -2.0
    lane = jax.lax.broadcasted_iota(jnp.int32, (R, T), 1)
    win = []
    for w in range(2):
        md = jnp.full((R, T), jnp.inf, jnp.float32)
        ti = jnp.zeros((R, T), jnp.int32)
        for t in range(w * (W // T), (w + 1) * (W // T)):
            cb_t = cb_ref[t * T:(t + 1) * T, :]
            e2_t = e2_ref[:, t * T:(t + 1) * T]
            sim2 = jax.lax.dot_general(
                xm2, cb_t, (((1,), (1,)), ((), ())),
                preferred_element_type=jnp.float32,
            )
            d = (x2 + e2_t) + sim2
            ti = jnp.where(d < md, t, ti)
            md = jnp.minimum(md, d)
        ci = ti * T + lane
        m = jnp.min(md, axis=1, keepdims=True)
        i = jnp.min(jnp.where(md == m, ci, K), axis=1, keepdims=True)
        win.append((m, i))
    (m1, i1), (m2, i2) = win
    v1b = (-m1).astype(jnp.bfloat16).astype(jnp.float32)
    idx = jnp.where((-m2) > v1b, i2, i1)
    ohl = (lane == (idx & (T - 1))).astype(jnp.bfloat16)
    tid = jax.lax.shift_right_logical(idx, TSHIFT)
    w2 = jax.lax.dot_general(
        ohl, cbh_ref[...], (((1,), (0,)), ((), ())),
        preferred_element_type=jnp.float32,
    ) + jax.lax.dot_general(
        ohl, cbl_ref[...], (((1,), (0,)), ((), ())),
        preferred_element_type=jnp.float32,
    )
    q = jnp.zeros((R, D), jnp.float32)
    for t in range(NT):
        q = jnp.where(tid == t, w2[:, t * D:(t + 1) * D], q)
    out_ref[...] = x + (q - x)


@jax.jit
def kernel(x, codebook):
    flat = jnp.reshape(x, (-1, D))
    n = flat.shape[0]
    x2 = jnp.sum(jnp.square(flat), axis=1, keepdims=True)
    e2 = jnp.sum(jnp.square(codebook), axis=1)[None, :]
    cbr = jnp.reshape(
        jnp.transpose(jnp.reshape(codebook, (NT, T, D)), (1, 0, 2)),
        (T, NT * D))
    cbh = cbr.astype(jnp.bfloat16)
    cbl = (cbr - cbh.astype(jnp.float32)).astype(jnp.bfloat16)
    y = pl.pallas_call(
        _vq_block,
        grid=(n // R,),
        in_specs=[
            pl.BlockSpec((R, D), lambda i: (i, 0)),
            pl.BlockSpec((K, D), lambda i: (0, 0)),
            pl.BlockSpec((R, 1), lambda i: (i, 0)),
            pl.BlockSpec((1, K), lambda i: (0, 0)),
            pl.BlockSpec((T, NT * D), lambda i: (0, 0)),
            pl.BlockSpec((T, NT * D), lambda i: (0, 0)),
        ],
        out_specs=pl.BlockSpec((R, D), lambda i: (i, 0)),
        out_shape=jax.ShapeDtypeStruct((n, D), jnp.float32),
    )(flat, codebook, x2, e2, cbh, cbl)
    return jnp.reshape(y, x.shape)


# speedup vs baseline: 1.6800x; 1.1204x over previous
import jax
import jax.numpy as jnp
from jax.experimental import pallas as pl

K = 8192
W = 4096
D = 32
R = 512
T = 1024
NT = K // T
TSHIFT = 10


def _vq_block(x_ref, cb_ref, x2_ref, e2_ref, cbh_ref, cbl_ref, out_ref):
    x = x_ref[...]
    x2 = x2_ref[...]
    xm2 = x * -2.0
    lane = jax.lax.broadcasted_iota(jnp.int32, (R, T), 1)
    win = []
    for w in range(2):
        md = jnp.full((R, T), jnp.inf, jnp.float32)
        ti = jnp.zeros((R, T), jnp.int32)
        for t in range(w * (W // T), (w + 1) * (W // T)):
            cb_t = cb_ref[t * T:(t + 1) * T, :]
            e2_t = e2_ref[:, t * T:(t + 1) * T]
            sim2 = jax.lax.dot_general(
                xm2, cb_t, (((1,), (1,)), ((), ())),
                preferred_element_type=jnp.float32,
            )
            d = (x2 + e2_t) + sim2
            ti = jnp.where(d < md, t, ti)
            md = jnp.minimum(md, d)
        ci = ti * T + lane
        m = jnp.min(md, axis=1, keepdims=True)
        i = jnp.min(jnp.where(md == m, ci, K), axis=1, keepdims=True)
        win.append((m, i))
    (m1, i1), (m2, i2) = win
    v1b = (-m1).astype(jnp.bfloat16).astype(jnp.float32)
    idx = jnp.where((-m2) > v1b, i2, i1)
    ohl = (lane == (idx & (T - 1))).astype(jnp.bfloat16)
    tid = jax.lax.shift_right_logical(idx, TSHIFT)
    w2 = jax.lax.dot_general(
        ohl, cbh_ref[...], (((1,), (0,)), ((), ())),
        preferred_element_type=jnp.float32,
    ) + jax.lax.dot_general(
        ohl, cbl_ref[...], (((1,), (0,)), ((), ())),
        preferred_element_type=jnp.float32,
    )
    q = jnp.zeros((R, D), jnp.float32)
    for t in range(NT):
        q = jnp.where(tid == t, w2[:, t * D:(t + 1) * D], q)
    out_ref[...] = x + (q - x)


@jax.jit
def kernel(x, codebook):
    flat = jnp.reshape(x, (-1, D))
    n = flat.shape[0]
    x2 = jnp.sum(jnp.square(flat), axis=1, keepdims=True)
    e2 = jnp.sum(jnp.square(codebook), axis=1)[None, :]
    cbr = jnp.reshape(
        jnp.transpose(jnp.reshape(codebook, (NT, T, D)), (1, 0, 2)),
        (T, NT * D))
    cbh = cbr.astype(jnp.bfloat16)
    cbl = (cbr - cbh.astype(jnp.float32)).astype(jnp.bfloat16)
    y = pl.pallas_call(
        _vq_block,
        grid=(n // R,),
        in_specs=[
            pl.BlockSpec((R, D), lambda i: (i, 0)),
            pl.BlockSpec((K, D), lambda i: (0, 0)),
            pl.BlockSpec((R, 1), lambda i: (i, 0)),
            pl.BlockSpec((1, K), lambda i: (0, 0)),
            pl.BlockSpec((T, NT * D), lambda i: (0, 0)),
            pl.BlockSpec((T, NT * D), lambda i: (0, 0)),
        ],
        out_specs=pl.BlockSpec((R, D), lambda i: (i, 0)),
        out_shape=jax.ShapeDtypeStruct((n, D), jnp.float32),
    )(flat, codebook, x2, e2, cbh, cbl)
    return jnp.reshape(y, x.shape)


# speedup vs baseline: 1.7097x; 1.0177x over previous
import jax
import jax.numpy as jnp
from jax.experimental import pallas as pl

K = 8192
W = 4096
D = 32
R = 1024
T = 1024
NT = K // T
TSHIFT = 10


def _vq_block(x_ref, cb_ref, x2_ref, e2_ref, cbh_ref, cbl_ref, out_ref):
    x = x_ref[...]
    x2 = x2_ref[...]
    xm2 = x * -2.0
    lane = jax.lax.broadcasted_iota(jnp.int32, (R, T), 1)
    win = []
    for w in range(2):
        md = jnp.full((R, T), jnp.inf, jnp.float32)
        ti = jnp.zeros((R, T), jnp.int32)
        for t in range(w * (W // T), (w + 1) * (W // T)):
            cb_t = cb_ref[t * T:(t + 1) * T, :]
            e2_t = e2_ref[:, t * T:(t + 1) * T]
            sim2 = jax.lax.dot_general(
                xm2, cb_t, (((1,), (1,)), ((), ())),
                preferred_element_type=jnp.float32,
            )
            d = (x2 + e2_t) + sim2
            ti = jnp.where(d < md, t, ti)
            md = jnp.minimum(md, d)
        ci = ti * T + lane
        m = jnp.min(md, axis=1, keepdims=True)
        i = jnp.min(jnp.where(md == m, ci, K), axis=1, keepdims=True)
        win.append((m, i))
    (m1, i1), (m2, i2) = win
    v1b = (-m1).astype(jnp.bfloat16).astype(jnp.float32)
    idx = jnp.where((-m2) > v1b, i2, i1)
    ohl = (lane == (idx & (T - 1))).astype(jnp.bfloat16)
    tid = jax.lax.shift_right_logical(idx, TSHIFT)
    w2 = jax.lax.dot_general(
        ohl, cbh_ref[...], (((1,), (0,)), ((), ())),
        preferred_element_type=jnp.float32,
    ) + jax.lax.dot_general(
        ohl, cbl_ref[...], (((1,), (0,)), ((), ())),
        preferred_element_type=jnp.float32,
    )
    q = jnp.zeros((R, D), jnp.float32)
    for t in range(NT):
        q = jnp.where(tid == t, w2[:, t * D:(t + 1) * D], q)
    out_ref[...] = x + (q - x)


@jax.jit
def kernel(x, codebook):
    flat = jnp.reshape(x, (-1, D))
    n = flat.shape[0]
    x2 = jnp.sum(jnp.square(flat), axis=1, keepdims=True)
    e2 = jnp.sum(jnp.square(codebook), axis=1)[None, :]
    cbr = jnp.reshape(
        jnp.transpose(jnp.reshape(codebook, (NT, T, D)), (1, 0, 2)),
        (T, NT * D))
    cbh = cbr.astype(jnp.bfloat16)
    cbl = (cbr - cbh.astype(jnp.float32)).astype(jnp.bfloat16)
    y = pl.pallas_call(
        _vq_block,
        grid=(n // R,),
        in_specs=[
            pl.BlockSpec((R, D), lambda i: (i, 0)),
            pl.BlockSpec((K, D), lambda i: (0, 0)),
            pl.BlockSpec((R, 1), lambda i: (i, 0)),
            pl.BlockSpec((1, K), lambda i: (0, 0)),
            pl.BlockSpec((T, NT * D), lambda i: (0, 0)),
            pl.BlockSpec((T, NT * D), lambda i: (0, 0)),
        ],
        out_specs=pl.BlockSpec((R, D), lambda i: (i, 0)),
        out_shape=jax.ShapeDtypeStruct((n, D), jnp.float32),
    )(flat, codebook, x2, e2, cbh, cbl)
    return jnp.reshape(y, x.shape)


# speedup vs baseline: 1.7512x; 1.0243x over previous
import jax
import jax.numpy as jnp
from jax.experimental import pallas as pl

K = 8192
W = 4096
D = 32
R = 2048
T = 1024
NT = K // T
TSHIFT = 10


def _vq_block(x_ref, cb_ref, x2_ref, e2_ref, cbh_ref, cbl_ref, out_ref):
    x = x_ref[...]
    x2 = x2_ref[...]
    xm2 = x * -2.0
    lane = jax.lax.broadcasted_iota(jnp.int32, (R, T), 1)
    win = []
    for w in range(2):
        md = jnp.full((R, T), jnp.inf, jnp.float32)
        ti = jnp.zeros((R, T), jnp.int32)
        for t in range(w * (W // T), (w + 1) * (W // T)):
            cb_t = cb_ref[t * T:(t + 1) * T, :]
            e2_t = e2_ref[:, t * T:(t + 1) * T]
            sim2 = jax.lax.dot_general(
                xm2, cb_t, (((1,), (1,)), ((), ())),
                preferred_element_type=jnp.float32,
            )
            d = (x2 + e2_t) + sim2
            ti = jnp.where(d < md, t, ti)
            md = jnp.minimum(md, d)
        ci = ti * T + lane
        m = jnp.min(md, axis=1, keepdims=True)
        i = jnp.min(jnp.where(md == m, ci, K), axis=1, keepdims=True)
        win.append((m, i))
    (m1, i1), (m2, i2) = win
    v1b = (-m1).astype(jnp.bfloat16).astype(jnp.float32)
    idx = jnp.where((-m2) > v1b, i2, i1)
    ohl = (lane == (idx & (T - 1))).astype(jnp.bfloat16)
    tid = jax.lax.shift_right_logical(idx, TSHIFT)
    w2 = jax.lax.dot_general(
        ohl, cbh_ref[...], (((1,), (0,)), ((), ())),
        preferred_element_type=jnp.float32,
    ) + jax.lax.dot_general(
        ohl, cbl_ref[...], (((1,), (0,)), ((), ())),
        preferred_element_type=jnp.float32,
    )
    q = jnp.zeros((R, D), jnp.float32)
    for t in range(NT):
        q = jnp.where(tid == t, w2[:, t * D:(t + 1) * D], q)
    out_ref[...] = x + (q - x)


@jax.jit
def kernel(x, codebook):
    flat = jnp.reshape(x, (-1, D))
    n = flat.shape[0]
    x2 = jnp.sum(jnp.square(flat), axis=1, keepdims=True)
    e2 = jnp.sum(jnp.square(codebook), axis=1)[None, :]
    cbr = jnp.reshape(
        jnp.transpose(jnp.reshape(codebook, (NT, T, D)), (1, 0, 2)),
        (T, NT * D))
    cbh = cbr.astype(jnp.bfloat16)
    cbl = (cbr - cbh.astype(jnp.float32)).astype(jnp.bfloat16)
    y = pl.pallas_call(
        _vq_block,
        grid=(n // R,),
        in_specs=[
            pl.BlockSpec((R, D), lambda i: (i, 0)),
            pl.BlockSpec((K, D), lambda i: (0, 0)),
            pl.BlockSpec((R, 1), lambda i: (i, 0)),
            pl.BlockSpec((1, K), lambda i: (0, 0)),
            pl.BlockSpec((T, NT * D), lambda i: (0, 0)),
            pl.BlockSpec((T, NT * D), lambda i: (0, 0)),
        ],
        out_specs=pl.BlockSpec((R, D), lambda i: (i, 0)),
        out_shape=jax.ShapeDtypeStruct((n, D), jnp.float32),
    )(flat, codebook, x2, e2, cbh, cbl)
    return jnp.reshape(y, x.shape)
